# Initial kernel scaffold; baseline (speedup 1.0000x reference)
import jax
import jax.numpy as jnp
from jax.experimental import pallas as pl


def kernel(x, fp, edge_index, batch, W_pre, b_pre, a_pre, Wl1, bl1, Wr1, a1, Wl2, bl2, Wr2, a2, W_fp, b_fp, a_fp, W_post, b_post):
    raise NotImplementedError("write your pallas kernel here")


# speedup vs baseline: 3.5765x; 3.5765x over previous
import functools

import jax
import jax.numpy as jnp
from jax import lax
from jax.experimental import pallas as pl
from jax.experimental.pallas import tpu as pltpu
from jax.experimental.pallas import tpu_sc as plsc

N = 10000
E = 640000
H = 128
G = 128
FP_DIM = 2048

NC = 2
NS = 16
CHUNK = 128
EDGE_ROWS = 5120
ROWS_PER_TILE = EDGE_ROWS // (NC * NS)
E_PAD = EDGE_ROWS * CHUNK

N_PAD = 10240
ZROWS = N_PAD // NS
BR = 1280
NB = N_PAD // BR


def _prelu(v, a):
    return jnp.where(v >= 0, v, a * v)


def _pre_body(x_ref, w_ref, b_ref, a_ref, o_ref):
    i = pl.program_id(0)
    v = jnp.dot(x_ref[...], w_ref[...], preferred_element_type=jnp.float32)
    v = _prelu(v + b_ref[...], a_ref[...])
    rows = lax.broadcasted_iota(jnp.int32, v.shape, 0) + i * BR
    o_ref[...] = jnp.where(rows < N, v, 0.0)


def _tc_pre(x_pad, W, b, a):
    return pl.pallas_call(
        _pre_body,
        grid=(NB,),
        in_specs=[
            pl.BlockSpec((BR, H), lambda i: (i, 0)),
            pl.BlockSpec((H, H), lambda i: (0, 0)),
            pl.BlockSpec((1, H), lambda i: (0, 0)),
            pl.BlockSpec((1, H), lambda i: (0, 0)),
        ],
        out_specs=pl.BlockSpec((BR, H), lambda i: (i, 0)),
        out_shape=jax.ShapeDtypeStruct((N_PAD, H), jnp.float32),
    )(x_pad, W, b, a)


def _combine_body(p_ref, h_ref, wl_ref, bl_ref, wr_ref, a_ref, o_ref):
    i = pl.program_id(0)
    agg = p_ref[0] + p_ref[1]
    v = jnp.dot(agg, wl_ref[...], preferred_element_type=jnp.float32)
    v += jnp.dot(h_ref[...], wr_ref[...], preferred_element_type=jnp.float32)
    v = _prelu(v + bl_ref[...], a_ref[...])
    rows = lax.broadcasted_iota(jnp.int32, v.shape, 0) + i * BR
    o_ref[...] = jnp.where(rows < N, v, 0.0)


def _tc_combine(P, h, Wl, bl, Wr, a):
    return pl.pallas_call(
        _combine_body,
        grid=(NB,),
        in_specs=[
            pl.BlockSpec((2, BR, H), lambda i: (0, i, 0)),
            pl.BlockSpec((BR, H), lambda i: (i, 0)),
            pl.BlockSpec((H, H), lambda i: (0, 0)),
            pl.BlockSpec((1, H), lambda i: (0, 0)),
            pl.BlockSpec((H, H), lambda i: (0, 0)),
            pl.BlockSpec((1, H), lambda i: (0, 0)),
        ],
        out_specs=pl.BlockSpec((BR, H), lambda i: (i, 0)),
        out_shape=jax.ShapeDtypeStruct((N_PAD, H), jnp.float32),
    )(P, h, Wl, bl, Wr, a)


def _tail_body(h_ref, b_ref, fp_ref, wfp_ref, bfp_ref, afp_ref,
               wpa_ref, wpb_ref, bp_ref, o_ref, acc, cnt):
    i = pl.program_id(0)

    @pl.when(i == 0)
    def _init():
        acc[...] = jnp.zeros((G, H), jnp.float32)
        cnt[...] = jnp.zeros((G, H), jnp.float32)

    bb = b_ref[0]
    oh = (bb[None, :] == lax.broadcasted_iota(jnp.int32, (G, BR), 0)
          ).astype(jnp.float32)
    acc[...] += jnp.dot(oh, h_ref[...], preferred_element_type=jnp.float32)
    cnt[...] += jnp.dot(oh, jnp.ones((BR, H), jnp.float32),
                        preferred_element_type=jnp.float32)

    @pl.when(i == NB - 1)
    def _fin():
        pooled = acc[...] / jnp.maximum(cnt[...], 1.0)
        fpe = jnp.dot(fp_ref[...], wfp_ref[...],
                      preferred_element_type=jnp.float32)
        fpe = _prelu(fpe + bfp_ref[...], afp_ref[...])
        out = jnp.dot(pooled, wpa_ref[...], preferred_element_type=jnp.float32)
        out += jnp.dot(fpe, wpb_ref[...], preferred_element_type=jnp.float32)
        o_ref[...] = out + bp_ref[...]


def _tc_tail(h2, batch2d, fp, W_fp, b_fp, a_fp, Wp_a, Wp_b, b_post):
    return pl.pallas_call(
        _tail_body,
        grid=(NB,),
        in_specs=[
            pl.BlockSpec((BR, H), lambda i: (i, 0)),
            pl.BlockSpec((1, BR), lambda i: (0, i)),
            pl.BlockSpec((G, FP_DIM), lambda i: (0, 0)),
            pl.BlockSpec((FP_DIM, H), lambda i: (0, 0)),
            pl.BlockSpec((1, H), lambda i: (0, 0)),
            pl.BlockSpec((1, H), lambda i: (0, 0)),
            pl.BlockSpec((H, H), lambda i: (0, 0)),
            pl.BlockSpec((H, H), lambda i: (0, 0)),
            pl.BlockSpec((1, H), lambda i: (0, 0)),
        ],
        out_specs=pl.BlockSpec((G, H), lambda i: (0, 0)),
        out_shape=jax.ShapeDtypeStruct((G, H), jnp.float32),
        scratch_shapes=[
            pltpu.VMEM((G, H), jnp.float32),
            pltpu.VMEM((G, H), jnp.float32),
        ],
    )(h2, batch2d, fp, W_fp, b_fp, a_fp, Wp_a, Wp_b, b_post)


IDX_BLK = 32


def _sc_agg_body(h_hbm, src_hbm, dst_hbm, out_hbm, idx_s, idx_d, rows, acc, sem):
    c = lax.axis_index("c")
    s = lax.axis_index("s")
    wid = c * NS + s

    zero16 = jnp.zeros((16,), jnp.float32)

    def _zrow(i, _):
        def _zcol(j, _):
            rows[i, pl.ds(j * 16, 16)] = zero16
            return 0
        return lax.fori_loop(0, H // 16, _zcol, 0)

    lax.fori_loop(0, CHUNK, _zrow, 0)
    base = s * ZROWS
    for k in range(ZROWS // CHUNK):
        pltpu.sync_copy(rows, acc.at[pl.ds(base + k * CHUNK, CHUNK)])
    plsc.subcore_barrier()

    def _blk(bi, _):
        off = wid * ROWS_PER_TILE + bi * IDX_BLK
        pltpu.sync_copy(src_hbm.at[pl.ds(off, IDX_BLK)], idx_s)
        pltpu.sync_copy(dst_hbm.at[pl.ds(off, IDX_BLK)], idx_d)

        def _edge_chunk(j, _):
            pltpu.async_copy(h_hbm.at[idx_s.at[j]], rows, sem).wait()
            pltpu.sync_copy(rows, acc.at[idx_d.at[j]], add=True)
            return 0

        lax.fori_loop(0, IDX_BLK, _edge_chunk, 0)
        return 0

    lax.fori_loop(0, ROWS_PER_TILE // IDX_BLK, _blk, 0)
    plsc.subcore_barrier()

    for k in range(ZROWS // CHUNK):
        off = base + k * CHUNK
        pltpu.sync_copy(acc.at[pl.ds(off, CHUNK)],
                        out_hbm.at[c, pl.ds(off, CHUNK)])


def _sc_agg(h_pad, src2d, dst2d):
    mesh = plsc.VectorSubcoreMesh(core_axis_name="c", subcore_axis_name="s",
                                  num_cores=NC, num_subcores=NS)
    f = pl.kernel(
        _sc_agg_body,
        jax.ShapeDtypeStruct((NC, N_PAD, H), jnp.float32),
        mesh=mesh,
        scratch_types=[
            pltpu.VMEM((IDX_BLK, CHUNK), jnp.int32),
            pltpu.VMEM((IDX_BLK, CHUNK), jnp.int32),
            pltpu.VMEM((CHUNK, H), jnp.float32),
            pltpu.VMEM_SHARED((N_PAD, H), jnp.float32),
            pltpu.SemaphoreType.DMA,
        ],
    )
    return f(h_pad, src2d, dst2d)


def kernel(x, fp, edge_index, batch, W_pre, b_pre, a_pre, Wl1, bl1, Wr1, a1,
           Wl2, bl2, Wr2, a2, W_fp, b_fp, a_fp, W_post, b_post):
    f32 = jnp.float32
    pad_idx = jnp.full((E_PAD - E,), N, jnp.int32)
    src2d = jnp.concatenate([edge_index[0], pad_idx]).reshape(EDGE_ROWS, CHUNK)
    dst2d = jnp.concatenate([edge_index[1], pad_idx]).reshape(EDGE_ROWS, CHUNK)
    x_pad = jnp.pad(x, ((0, N_PAD - N), (0, 0)))
    batch2d = jnp.pad(batch, (0, N_PAD - N), constant_values=G).reshape(1, N_PAD)
    b_pre2 = b_pre.reshape(1, H)
    a_pre2 = a_pre.reshape(1, H)
    bl1_2, a1_2 = bl1.reshape(1, H), a1.reshape(1, H)
    bl2_2, a2_2 = bl2.reshape(1, H), a2.reshape(1, H)
    b_fp2, a_fp2 = b_fp.reshape(1, H), a_fp.reshape(1, H)
    b_post2 = b_post.reshape(1, H)
    Wp_a, Wp_b = W_post[:H], W_post[H:]

    h0 = _tc_pre(x_pad.astype(f32), W_pre, b_pre2, a_pre2)
    P1 = _sc_agg(h0, src2d, dst2d)
    h1 = _tc_combine(P1, h0, Wl1, bl1_2, Wr1, a1_2)
    P2 = _sc_agg(h1, src2d, dst2d)
    h2 = _tc_combine(P2, h1, Wl2, bl2_2, Wr2, a2_2)
    return _tc_tail(h2, batch2d, fp, W_fp, b_fp2, a_fp2, Wp_a, Wp_b, b_post2)


# speedup vs baseline: 3.9086x; 1.0928x over previous
import functools

import jax
import jax.numpy as jnp
from jax import lax
from jax.experimental import pallas as pl
from jax.experimental.pallas import tpu as pltpu
from jax.experimental.pallas import tpu_sc as plsc

N = 10000
E = 640000
H = 128
G = 128
FP_DIM = 2048

NC = 2
NS = 16
CHUNK = 128
EDGE_ROWS = 5120
ROWS_PER_TILE = EDGE_ROWS // (NC * NS)
E_PAD = EDGE_ROWS * CHUNK

N_PAD = 10240
ZROWS = N_PAD // NS
BR = 1280
NB = N_PAD // BR


def _prelu(v, a):
    return jnp.where(v >= 0, v, a * v)


def _pre_body(x_ref, w_ref, b_ref, a_ref, o_ref):
    i = pl.program_id(0)
    v = jnp.dot(x_ref[...], w_ref[...], preferred_element_type=jnp.float32)
    v = _prelu(v + b_ref[...], a_ref[...])
    rows = lax.broadcasted_iota(jnp.int32, v.shape, 0) + i * BR
    o_ref[...] = jnp.where(rows < N, v, 0.0)


def _tc_pre(x_pad, W, b, a):
    return pl.pallas_call(
        _pre_body,
        grid=(NB,),
        in_specs=[
            pl.BlockSpec((BR, H), lambda i: (i, 0)),
            pl.BlockSpec((H, H), lambda i: (0, 0)),
            pl.BlockSpec((1, H), lambda i: (0, 0)),
            pl.BlockSpec((1, H), lambda i: (0, 0)),
        ],
        out_specs=pl.BlockSpec((BR, H), lambda i: (i, 0)),
        out_shape=jax.ShapeDtypeStruct((N_PAD, H), jnp.float32),
    )(x_pad, W, b, a)


def _combine_body(p_ref, h_ref, wl_ref, bl_ref, wr_ref, a_ref, o_ref):
    i = pl.program_id(0)
    agg = p_ref[0] + p_ref[1]
    v = jnp.dot(agg, wl_ref[...], preferred_element_type=jnp.float32)
    v += jnp.dot(h_ref[...], wr_ref[...], preferred_element_type=jnp.float32)
    v = _prelu(v + bl_ref[...], a_ref[...])
    rows = lax.broadcasted_iota(jnp.int32, v.shape, 0) + i * BR
    o_ref[...] = jnp.where(rows < N, v, 0.0)


def _tc_combine(P, h, Wl, bl, Wr, a):
    return pl.pallas_call(
        _combine_body,
        grid=(NB,),
        in_specs=[
            pl.BlockSpec((2, BR, H), lambda i: (0, i, 0)),
            pl.BlockSpec((BR, H), lambda i: (i, 0)),
            pl.BlockSpec((H, H), lambda i: (0, 0)),
            pl.BlockSpec((1, H), lambda i: (0, 0)),
            pl.BlockSpec((H, H), lambda i: (0, 0)),
            pl.BlockSpec((1, H), lambda i: (0, 0)),
        ],
        out_specs=pl.BlockSpec((BR, H), lambda i: (i, 0)),
        out_shape=jax.ShapeDtypeStruct((N_PAD, H), jnp.float32),
    )(P, h, Wl, bl, Wr, a)


def _tail_body(h_ref, b_ref, fp_ref, wfp_ref, bfp_ref, afp_ref,
               wpa_ref, wpb_ref, bp_ref, o_ref, acc, cnt):
    i = pl.program_id(0)

    @pl.when(i == 0)
    def _init():
        acc[...] = jnp.zeros((G, H), jnp.float32)
        cnt[...] = jnp.zeros((G, H), jnp.float32)

    bb = b_ref[0]
    oh = (bb[None, :] == lax.broadcasted_iota(jnp.int32, (G, BR), 0)
          ).astype(jnp.float32)
    acc[...] += jnp.dot(oh, h_ref[...], preferred_element_type=jnp.float32)
    cnt[...] += jnp.dot(oh, jnp.ones((BR, H), jnp.float32),
                        preferred_element_type=jnp.float32)

    @pl.when(i == NB - 1)
    def _fin():
        pooled = acc[...] / jnp.maximum(cnt[...], 1.0)
        fpe = jnp.dot(fp_ref[...], wfp_ref[...],
                      preferred_element_type=jnp.float32)
        fpe = _prelu(fpe + bfp_ref[...], afp_ref[...])
        out = jnp.dot(pooled, wpa_ref[...], preferred_element_type=jnp.float32)
        out += jnp.dot(fpe, wpb_ref[...], preferred_element_type=jnp.float32)
        o_ref[...] = out + bp_ref[...]


def _tc_tail(h2, batch2d, fp, W_fp, b_fp, a_fp, Wp_a, Wp_b, b_post):
    return pl.pallas_call(
        _tail_body,
        grid=(NB,),
        in_specs=[
            pl.BlockSpec((BR, H), lambda i: (i, 0)),
            pl.BlockSpec((1, BR), lambda i: (0, i)),
            pl.BlockSpec((G, FP_DIM), lambda i: (0, 0)),
            pl.BlockSpec((FP_DIM, H), lambda i: (0, 0)),
            pl.BlockSpec((1, H), lambda i: (0, 0)),
            pl.BlockSpec((1, H), lambda i: (0, 0)),
            pl.BlockSpec((H, H), lambda i: (0, 0)),
            pl.BlockSpec((H, H), lambda i: (0, 0)),
            pl.BlockSpec((1, H), lambda i: (0, 0)),
        ],
        out_specs=pl.BlockSpec((G, H), lambda i: (0, 0)),
        out_shape=jax.ShapeDtypeStruct((G, H), jnp.float32),
        scratch_shapes=[
            pltpu.VMEM((G, H), jnp.float32),
            pltpu.VMEM((G, H), jnp.float32),
        ],
    )(h2, batch2d, fp, W_fp, b_fp, a_fp, Wp_a, Wp_b, b_post)


IDX_BLK = 16


def _sc_agg_body(h_hbm, src_hbm, dst_hbm, out_hbm, idx_s, idx_d,
                 rows0, rows1, acc, sem_g0, sem_g1):
    c = lax.axis_index("c")
    s = lax.axis_index("s")
    wid = c * NS + s

    zero16 = jnp.zeros((16,), jnp.float32)

    def _zrow(i, _):
        def _zcol(j, _):
            rows0[i, pl.ds(j * 16, 16)] = zero16
            return 0
        return lax.fori_loop(0, H // 16, _zcol, 0)

    lax.fori_loop(0, CHUNK, _zrow, 0)
    base = s * ZROWS
    for k in range(ZROWS // CHUNK):
        pltpu.sync_copy(rows0, acc.at[pl.ds(base + k * CHUNK, CHUNK)])
    plsc.subcore_barrier()

    buf = [(rows0, sem_g0), (rows1, sem_g1)]

    def _blk(bi, _):
        off = wid * ROWS_PER_TILE + bi * IDX_BLK
        pltpu.sync_copy(src_hbm.at[pl.ds(off, IDX_BLK)], idx_s)
        pltpu.sync_copy(dst_hbm.at[pl.ds(off, IDX_BLK)], idx_d)
        pltpu.async_copy(h_hbm.at[idx_s.at[0]], rows0, sem_g0)
        for j in range(IDX_BLK):
            rp, sp = buf[j % 2]
            if j + 1 < IDX_BLK:
                rq, sq = buf[(j + 1) % 2]
                pltpu.async_copy(h_hbm.at[idx_s.at[j + 1]], rq, sq)
            pltpu.make_async_copy(h_hbm.at[idx_s.at[j]], rp, sp).wait()
            pltpu.sync_copy(rp, acc.at[idx_d.at[j]], add=True)
        return 0

    lax.fori_loop(0, ROWS_PER_TILE // IDX_BLK, _blk, 0)
    plsc.subcore_barrier()

    for k in range(ZROWS // CHUNK):
        off = base + k * CHUNK
        pltpu.sync_copy(acc.at[pl.ds(off, CHUNK)],
                        out_hbm.at[c, pl.ds(off, CHUNK)])


def _sc_agg(h_pad, src2d, dst2d):
    mesh = plsc.VectorSubcoreMesh(core_axis_name="c", subcore_axis_name="s",
                                  num_cores=NC, num_subcores=NS)
    f = pl.kernel(
        _sc_agg_body,
        jax.ShapeDtypeStruct((NC, N_PAD, H), jnp.float32),
        mesh=mesh,
        scratch_types=[
            pltpu.VMEM((IDX_BLK, CHUNK), jnp.int32),
            pltpu.VMEM((IDX_BLK, CHUNK), jnp.int32),
            pltpu.VMEM((CHUNK, H), jnp.float32),
            pltpu.VMEM((CHUNK, H), jnp.float32),
            pltpu.VMEM_SHARED((N_PAD, H), jnp.float32),
            pltpu.SemaphoreType.DMA,
            pltpu.SemaphoreType.DMA,
        ],
    )
    return f(h_pad, src2d, dst2d)


def kernel(x, fp, edge_index, batch, W_pre, b_pre, a_pre, Wl1, bl1, Wr1, a1,
           Wl2, bl2, Wr2, a2, W_fp, b_fp, a_fp, W_post, b_post):
    f32 = jnp.float32
    pad_idx = jnp.full((E_PAD - E,), N, jnp.int32)
    src2d = jnp.concatenate([edge_index[0], pad_idx]).reshape(EDGE_ROWS, CHUNK)
    dst2d = jnp.concatenate([edge_index[1], pad_idx]).reshape(EDGE_ROWS, CHUNK)
    x_pad = jnp.pad(x, ((0, N_PAD - N), (0, 0)))
    batch2d = jnp.pad(batch, (0, N_PAD - N), constant_values=G).reshape(1, N_PAD)
    b_pre2 = b_pre.reshape(1, H)
    a_pre2 = a_pre.reshape(1, H)
    bl1_2, a1_2 = bl1.reshape(1, H), a1.reshape(1, H)
    bl2_2, a2_2 = bl2.reshape(1, H), a2.reshape(1, H)
    b_fp2, a_fp2 = b_fp.reshape(1, H), a_fp.reshape(1, H)
    b_post2 = b_post.reshape(1, H)
    Wp_a, Wp_b = W_post[:H], W_post[H:]

    h0 = _tc_pre(x_pad.astype(f32), W_pre, b_pre2, a_pre2)
    P1 = _sc_agg(h0, src2d, dst2d)
    h1 = _tc_combine(P1, h0, Wl1, bl1_2, Wr1, a1_2)
    P2 = _sc_agg(h1, src2d, dst2d)
    h2 = _tc_combine(P2, h1, Wl2, bl2_2, Wr2, a2_2)
    return _tc_tail(h2, batch2d, fp, W_fp, b_fp2, a_fp2, Wp_a, Wp_b, b_post2)


# speedup vs baseline: 4.2369x; 1.0840x over previous
import functools

import jax
import jax.numpy as jnp
from jax import lax
from jax.experimental import pallas as pl
from jax.experimental.pallas import tpu as pltpu
from jax.experimental.pallas import tpu_sc as plsc

N = 10000
E = 640000
H = 128
G = 128
FP_DIM = 2048

NC = 2
NS = 16
CHUNK = 128
EDGE_ROWS = 5120
ROWS_PER_TILE = EDGE_ROWS // (NC * NS)
E_PAD = EDGE_ROWS * CHUNK

N_PAD = 10240
ZROWS = N_PAD // NS
BR = 1280
NB = N_PAD // BR


def _prelu(v, a):
    return jnp.where(v >= 0, v, a * v)


def _pre_body(x_ref, w_ref, b_ref, a_ref, o_ref):
    i = pl.program_id(0)
    v = jnp.dot(x_ref[...], w_ref[...], preferred_element_type=jnp.float32)
    v = _prelu(v + b_ref[...], a_ref[...])
    rows = lax.broadcasted_iota(jnp.int32, v.shape, 0) + i * BR
    o_ref[...] = jnp.where(rows < N, v, 0.0)


def _tc_pre(x_pad, W, b, a):
    return pl.pallas_call(
        _pre_body,
        grid=(NB,),
        in_specs=[
            pl.BlockSpec((BR, H), lambda i: (i, 0)),
            pl.BlockSpec((H, H), lambda i: (0, 0)),
            pl.BlockSpec((1, H), lambda i: (0, 0)),
            pl.BlockSpec((1, H), lambda i: (0, 0)),
        ],
        out_specs=pl.BlockSpec((BR, H), lambda i: (i, 0)),
        out_shape=jax.ShapeDtypeStruct((N_PAD, H), jnp.float32),
    )(x_pad, W, b, a)


def _combine_body(p_ref, h_ref, wl_ref, bl_ref, wr_ref, a_ref, o_ref):
    i = pl.program_id(0)
    agg = p_ref[0] + p_ref[1]
    v = jnp.dot(agg, wl_ref[...], preferred_element_type=jnp.float32)
    v += jnp.dot(h_ref[...], wr_ref[...], preferred_element_type=jnp.float32)
    v = _prelu(v + bl_ref[...], a_ref[...])
    rows = lax.broadcasted_iota(jnp.int32, v.shape, 0) + i * BR
    o_ref[...] = jnp.where(rows < N, v, 0.0)


def _tc_combine(P, h, Wl, bl, Wr, a):
    return pl.pallas_call(
        _combine_body,
        grid=(NB,),
        in_specs=[
            pl.BlockSpec((2, BR, H), lambda i: (0, i, 0)),
            pl.BlockSpec((BR, H), lambda i: (i, 0)),
            pl.BlockSpec((H, H), lambda i: (0, 0)),
            pl.BlockSpec((1, H), lambda i: (0, 0)),
            pl.BlockSpec((H, H), lambda i: (0, 0)),
            pl.BlockSpec((1, H), lambda i: (0, 0)),
        ],
        out_specs=pl.BlockSpec((BR, H), lambda i: (i, 0)),
        out_shape=jax.ShapeDtypeStruct((N_PAD, H), jnp.float32),
    )(P, h, Wl, bl, Wr, a)


def _tail_body(h_ref, b_ref, fp_ref, wfp_ref, bfp_ref, afp_ref,
               wpa_ref, wpb_ref, bp_ref, o_ref, acc, cnt):
    i = pl.program_id(0)

    @pl.when(i == 0)
    def _init():
        acc[...] = jnp.zeros((G, H), jnp.float32)
        cnt[...] = jnp.zeros((G, H), jnp.float32)

    bb = b_ref[0]
    oh = (bb[None, :] == lax.broadcasted_iota(jnp.int32, (G, BR), 0)
          ).astype(jnp.float32)
    acc[...] += jnp.dot(oh, h_ref[...], preferred_element_type=jnp.float32)
    cnt[...] += jnp.dot(oh, jnp.ones((BR, H), jnp.float32),
                        preferred_element_type=jnp.float32)

    @pl.when(i == NB - 1)
    def _fin():
        pooled = acc[...] / jnp.maximum(cnt[...], 1.0)
        fpe = jnp.dot(fp_ref[...], wfp_ref[...],
                      preferred_element_type=jnp.float32)
        fpe = _prelu(fpe + bfp_ref[...], afp_ref[...])
        out = jnp.dot(pooled, wpa_ref[...], preferred_element_type=jnp.float32)
        out += jnp.dot(fpe, wpb_ref[...], preferred_element_type=jnp.float32)
        o_ref[...] = out + bp_ref[...]


def _tc_tail(h2, batch2d, fp, W_fp, b_fp, a_fp, Wp_a, Wp_b, b_post):
    return pl.pallas_call(
        _tail_body,
        grid=(NB,),
        in_specs=[
            pl.BlockSpec((BR, H), lambda i: (i, 0)),
            pl.BlockSpec((1, BR), lambda i: (0, i)),
            pl.BlockSpec((G, FP_DIM), lambda i: (0, 0)),
            pl.BlockSpec((FP_DIM, H), lambda i: (0, 0)),
            pl.BlockSpec((1, H), lambda i: (0, 0)),
            pl.BlockSpec((1, H), lambda i: (0, 0)),
            pl.BlockSpec((H, H), lambda i: (0, 0)),
            pl.BlockSpec((H, H), lambda i: (0, 0)),
            pl.BlockSpec((1, H), lambda i: (0, 0)),
        ],
        out_specs=pl.BlockSpec((G, H), lambda i: (0, 0)),
        out_shape=jax.ShapeDtypeStruct((G, H), jnp.float32),
        scratch_shapes=[
            pltpu.VMEM((G, H), jnp.float32),
            pltpu.VMEM((G, H), jnp.float32),
        ],
    )(h2, batch2d, fp, W_fp, b_fp, a_fp, Wp_a, Wp_b, b_post)


IDX_BLK = 16

R0 = 240
R1 = 80


def _sc_agg_body(h_hbm, src_hbm, dst_hbm, out_hbm, idx_s, idx_d,
                 rows0, rows1, acc, sem_g0, sem_g1):
    c = lax.axis_index("c")
    s = lax.axis_index("s")
    nrows = jnp.where(c == 0, R0, R1)
    start = pl.multiple_of(c * (NS * R0) + s * nrows, 8)

    zero16 = jnp.zeros((16,), jnp.float32)

    def _zrow(i, _):
        def _zcol(j, _):
            rows0[i, pl.ds(j * 16, 16)] = zero16
            return 0
        return lax.fori_loop(0, H // 16, _zcol, 0)

    lax.fori_loop(0, CHUNK, _zrow, 0)
    base = s * ZROWS
    for k in range(ZROWS // CHUNK):
        pltpu.sync_copy(rows0, acc.at[pl.ds(base + k * CHUNK, CHUNK)])
    plsc.subcore_barrier()

    buf = [(rows0, sem_g0), (rows1, sem_g1)]

    def _blk(bi, _):
        off = pl.multiple_of(start + bi * IDX_BLK, 8)
        pltpu.sync_copy(src_hbm.at[pl.ds(off, IDX_BLK)], idx_s)
        pltpu.sync_copy(dst_hbm.at[pl.ds(off, IDX_BLK)], idx_d)
        pltpu.async_copy(h_hbm.at[idx_s.at[0]], rows0, sem_g0)
        for j in range(IDX_BLK):
            rp, sp = buf[j % 2]
            if j + 1 < IDX_BLK:
                rq, sq = buf[(j + 1) % 2]
                pltpu.async_copy(h_hbm.at[idx_s.at[j + 1]], rq, sq)
            pltpu.make_async_copy(h_hbm.at[idx_s.at[j]], rp, sp).wait()
            pltpu.sync_copy(rp, acc.at[idx_d.at[j]], add=True)
        return 0

    lax.fori_loop(0, nrows // IDX_BLK, _blk, 0)
    plsc.subcore_barrier()

    for k in range(ZROWS // CHUNK):
        off = base + k * CHUNK
        pltpu.sync_copy(acc.at[pl.ds(off, CHUNK)],
                        out_hbm.at[c, pl.ds(off, CHUNK)])


def _sc_agg(h_pad, src2d, dst2d):
    mesh = plsc.VectorSubcoreMesh(core_axis_name="c", subcore_axis_name="s",
                                  num_cores=NC, num_subcores=NS)
    f = pl.kernel(
        _sc_agg_body,
        jax.ShapeDtypeStruct((NC, N_PAD, H), jnp.float32),
        mesh=mesh,
        scratch_types=[
            pltpu.VMEM((IDX_BLK, CHUNK), jnp.int32),
            pltpu.VMEM((IDX_BLK, CHUNK), jnp.int32),
            pltpu.VMEM((CHUNK, H), jnp.float32),
            pltpu.VMEM((CHUNK, H), jnp.float32),
            pltpu.VMEM_SHARED((N_PAD, H), jnp.float32),
            pltpu.SemaphoreType.DMA,
            pltpu.SemaphoreType.DMA,
        ],
    )
    return f(h_pad, src2d, dst2d)


def kernel(x, fp, edge_index, batch, W_pre, b_pre, a_pre, Wl1, bl1, Wr1, a1,
           Wl2, bl2, Wr2, a2, W_fp, b_fp, a_fp, W_post, b_post):
    f32 = jnp.float32
    pad_idx = jnp.full((E_PAD - E,), N, jnp.int32)
    src2d = jnp.concatenate([edge_index[0], pad_idx]).reshape(EDGE_ROWS, CHUNK)
    dst2d = jnp.concatenate([edge_index[1], pad_idx]).reshape(EDGE_ROWS, CHUNK)
    x_pad = jnp.pad(x, ((0, N_PAD - N), (0, 0)))
    batch2d = jnp.pad(batch, (0, N_PAD - N), constant_values=G).reshape(1, N_PAD)
    b_pre2 = b_pre.reshape(1, H)
    a_pre2 = a_pre.reshape(1, H)
    bl1_2, a1_2 = bl1.reshape(1, H), a1.reshape(1, H)
    bl2_2, a2_2 = bl2.reshape(1, H), a2.reshape(1, H)
    b_fp2, a_fp2 = b_fp.reshape(1, H), a_fp.reshape(1, H)
    b_post2 = b_post.reshape(1, H)
    Wp_a, Wp_b = W_post[:H], W_post[H:]

    h0 = _tc_pre(x_pad.astype(f32), W_pre, b_pre2, a_pre2)
    P1 = _sc_agg(h0, src2d, dst2d)
    h1 = _tc_combine(P1, h0, Wl1, bl1_2, Wr1, a1_2)
    P2 = _sc_agg(h1, src2d, dst2d)
    h2 = _tc_combine(P2, h1, Wl2, bl2_2, Wr2, a2_2)
    return _tc_tail(h2, batch2d, fp, W_fp, b_fp2, a_fp2, Wp_a, Wp_b, b_post2)
